# Initial kernel scaffold; baseline (speedup 1.0000x reference)
import jax
import jax.numpy as jnp
from jax.experimental import pallas as pl


def kernel(x, edge_index, edge_attr, Wn0, We0, A1w0, A1b0, A2w0, A2b0, b0, g0, bt0, Wn1, We1, A1w1, A1b1, A2w1, A2b1, b1, g1, bt1, Wn2, We2, A1w2, A1b2, A2w2, A2b2, b2, g2, bt2):
    raise NotImplementedError("write your pallas kernel here")


# speedup vs baseline: 2.8007x; 2.8007x over previous
import functools

import jax
import jax.numpy as jnp
from jax import lax
from jax.experimental import pallas as pl
from jax.experimental.pallas import tpu as pltpu
from jax.experimental.pallas import tpu_sc as plsc

N = 10000
E = 320000
D = 128
DH = 64

NC = 2
NS = 16
NW = NC * NS
EPW = E // NW
C = 200
NCHUNK = EPW // C
CS = 200
NCHUNK_S = EPW // CS
RPS = N // NS

_mesh = plsc.VectorSubcoreMesh(core_axis_name="c", subcore_axis_name="s")


@functools.partial(
    pl.kernel,
    mesh=_mesh,
    out_type=[
        jax.ShapeDtypeStruct((E, DH), jnp.float32),
        jax.ShapeDtypeStruct((E, D), jnp.float32),
    ],
    scratch_types=[
        pltpu.VMEM((C,), jnp.int32),
        pltpu.VMEM((C,), jnp.int32),
        pltpu.VMEM((C, D), jnp.float32),
        pltpu.VMEM((C, D), jnp.float32),
        pltpu.VMEM((C, D), jnp.float32),
        pltpu.VMEM((C, DH), jnp.float32),
        pltpu.SemaphoreType.DMA,
        pltpu.SemaphoreType.DMA,
        pltpu.SemaphoreType.DMA,
    ],
)
def _gather_k(p_hbm, m_hbm, src_hbm, dst_hbm, s_out, ms_out,
              idxs_v, idxd_v, bufd, bufs, bufm, sbuf, sem1, sem2, sem3):
    wid = lax.axis_index("s") * NC + lax.axis_index("c")
    base = wid * EPW

    def chunk(k, carry):
        off = base + k * C
        pltpu.sync_copy(src_hbm.at[pl.ds(off, C)], idxs_v)
        pltpu.sync_copy(dst_hbm.at[pl.ds(off, C)], idxd_v)
        cp1 = pltpu.async_copy(p_hbm.at[idxd_v], bufd, sem1)
        cp2 = pltpu.async_copy(p_hbm.at[idxs_v], bufs, sem2)
        cp3 = pltpu.async_copy(m_hbm.at[idxs_v], bufm, sem3)
        cp1.wait()
        cp2.wait()
        cp3.wait()

        def addrow(r, c2):
            for j in range(DH // 16):
                sbuf[r, pl.ds(j * 16, 16)] = (bufd[r, pl.ds(j * 16, 16)]
                                              + bufs[r, pl.ds(DH + j * 16, 16)])
            return c2

        lax.fori_loop(0, C, addrow, 0)
        pltpu.sync_copy(sbuf, s_out.at[pl.ds(off, C)])
        pltpu.sync_copy(bufm, ms_out.at[pl.ds(off, C)])
        return carry

    lax.fori_loop(0, NCHUNK, chunk, 0)


@functools.partial(
    pl.kernel,
    mesh=_mesh,
    out_type=jax.ShapeDtypeStruct((2 * N, D), jnp.float32),
    scratch_types=[
        pltpu.VMEM((CS,), jnp.int32),
        pltpu.VMEM((CS, D), jnp.float32),
        pltpu.VMEM_SHARED((N, D), jnp.float32),
    ],
)
def _scatter_k(msg_hbm, dst_hbm, out_hbm, idx_v, buf, acc):
    cid = lax.axis_index("c")
    sid = lax.axis_index("s")
    wid = sid * NC + cid

    def zrow(r, carry):
        for j in range(D // 16):
            buf[r, pl.ds(j * 16, 16)] = jnp.zeros((16,), jnp.float32)
        return carry

    lax.fori_loop(0, CS, zrow, 0)
    nzc = N // CS
    for tt in range(-(-nzc // NS)):
        t = tt * NS + sid

        def zcopy(tv=t):
            pltpu.sync_copy(buf, acc.at[pl.ds(tv * CS, CS)])

        pl.when(t < nzc)(zcopy)
    plsc.subcore_barrier()

    base = wid * EPW

    def chunk(k, carry):
        off = base + k * CS
        pltpu.sync_copy(dst_hbm.at[pl.ds(off, CS)], idx_v)
        pltpu.sync_copy(msg_hbm.at[pl.ds(off, CS)], buf)
        pltpu.sync_copy(buf, acc.at[idx_v], add=True)
        return carry

    lax.fori_loop(0, NCHUNK_S, chunk, 0)
    plsc.subcore_barrier()

    for tt in range(-(-nzc // NS)):
        t = tt * NS + sid

        def dcopy(tv=t):
            pltpu.sync_copy(acc.at[pl.ds(tv * CS, CS)],
                            out_hbm.at[pl.ds(cid * N + tv * CS, CS)])

        pl.when(t < nzc)(dcopy)


_NB = 400
_NGRID = N // _NB
_EB = 1600
_EGRID = E // _EB


def _node_body(h_ref, wn_ref, a1_ref, a1bias_ref, wet_ref, p_ref, m_ref):
    hn = jnp.dot(h_ref[...], wn_ref[...], preferred_element_type=jnp.float32)
    pa = jnp.dot(hn, a1_ref[...], preferred_element_type=jnp.float32)
    p_ref[...] = pa + a1bias_ref[...]
    m_ref[...] = jnp.dot(hn, wet_ref[...], preferred_element_type=jnp.float32)


_node_call = pl.pallas_call(
    _node_body,
    grid=(_NGRID,),
    in_specs=[
        pl.BlockSpec((_NB, D), lambda i: (i, 0)),
        pl.BlockSpec((D, D), lambda i: (0, 0)),
        pl.BlockSpec((D, D), lambda i: (0, 0)),
        pl.BlockSpec((1, D), lambda i: (0, 0)),
        pl.BlockSpec((D, D), lambda i: (0, 0)),
    ],
    out_specs=[
        pl.BlockSpec((_NB, D), lambda i: (i, 0)),
        pl.BlockSpec((_NB, D), lambda i: (i, 0)),
    ],
    out_shape=[
        jax.ShapeDtypeStruct((N, D), jnp.float32),
        jax.ShapeDtypeStruct((N, D), jnp.float32),
    ],
)


def _edge_body(s_ref, ms_ref, ea_ref, a2w_ref, a2b_ref, web_ref, msg_ref):
    srelu = jnp.maximum(s_ref[...], 0.0)
    z = jnp.sum(srelu * a2w_ref[...], axis=-1, keepdims=True) + a2b_ref[0, 0]
    att = jax.nn.sigmoid(z)
    ec = (ea_ref[:, 0:1] * web_ref[0:1, :] + ea_ref[:, 1:2] * web_ref[1:2, :])
    msg_ref[...] = att * (ms_ref[...] + ec)


_edge_call = pl.pallas_call(
    _edge_body,
    grid=(_EGRID,),
    in_specs=[
        pl.BlockSpec((_EB, DH), lambda i: (i, 0)),
        pl.BlockSpec((_EB, D), lambda i: (i, 0)),
        pl.BlockSpec((_EB, 2), lambda i: (i, 0)),
        pl.BlockSpec((1, DH), lambda i: (0, 0)),
        pl.BlockSpec((1, 1), lambda i: (0, 0)),
        pl.BlockSpec((2, D), lambda i: (0, 0)),
    ],
    out_specs=pl.BlockSpec((_EB, D), lambda i: (i, 0)),
    out_shape=jax.ShapeDtypeStruct((E, D), jnp.float32),
)


def _post_body(p0_ref, p1_ref, hin_ref, b_ref, g_ref, bt_ref, o_ref,
               *, apply_relu):
    t = p0_ref[...] + p1_ref[...] + b_ref[...]
    mu = jnp.mean(t, axis=-1, keepdims=True)
    var = jnp.mean((t - mu) ** 2, axis=-1, keepdims=True)
    y = (t - mu) * lax.rsqrt(var + 1e-5) * g_ref[...] + bt_ref[...]
    if apply_relu:
        y = jnp.maximum(y, 0.0)
    o_ref[...] = y + hin_ref[...]


def _post_call(apply_relu):
    return pl.pallas_call(
        functools.partial(_post_body, apply_relu=apply_relu),
        grid=(_NGRID,),
        in_specs=[
            pl.BlockSpec((_NB, D), lambda i: (i, 0)),
            pl.BlockSpec((_NB, D), lambda i: (i + _NGRID, 0)),
            pl.BlockSpec((_NB, D), lambda i: (i, 0)),
            pl.BlockSpec((1, D), lambda i: (0, 0)),
            pl.BlockSpec((1, D), lambda i: (0, 0)),
            pl.BlockSpec((1, D), lambda i: (0, 0)),
        ],
        out_specs=pl.BlockSpec((_NB, D), lambda i: (i, 0)),
        out_shape=jax.ShapeDtypeStruct((N, D), jnp.float32),
    )


def kernel(x, edge_index, edge_attr,
           Wn0, We0, A1w0, A1b0, A2w0, A2b0, b0, g0, bt0,
           Wn1, We1, A1w1, A1b1, A2w1, A2b1, b1, g1, bt1,
           Wn2, We2, A1w2, A1b2, A2w2, A2b2, b2, g2, bt2):
    src = edge_index[0]
    dst = edge_index[1]
    layers = [
        (Wn0, We0, A1w0, A1b0, A2w0, A2b0, b0, g0, bt0),
        (Wn1, We1, A1w1, A1b1, A2w1, A2b1, b1, g1, bt1),
        (Wn2, We2, A1w2, A1b2, A2w2, A2b2, b2, g2, bt2),
    ]
    h = x
    for li, (Wn, We, A1w, A1b, A2w, A2b, b, g, bt) in enumerate(layers):
        a1_comb = jnp.concatenate([A1w[:D], A1w[D:]], axis=1)
        a1bias = jnp.concatenate([A1b, jnp.zeros((DH,), A1b.dtype)])
        p, m = _node_call(h, Wn, a1_comb, a1bias.reshape(1, D), We[:D])
        s, ms = _gather_k(p, m, src, dst)
        msg = _edge_call(s, ms, edge_attr, A2w.reshape(1, DH),
                         A2b.reshape(1, 1), We[D:])
        part = _scatter_k(msg, dst)
        h = _post_call(li < 2)(part, part, h, b.reshape(1, D), g.reshape(1, D),
                               bt.reshape(1, D))
    return h
